# Initial kernel scaffold; baseline (speedup 1.0000x reference)
import jax
import jax.numpy as jnp
from jax.experimental import pallas as pl


def kernel(x, edge_index, edge_attr, W_rel, b_rel, W_root, gamma, beta, W1, b1, W2, b2):
    raise NotImplementedError("write your pallas kernel here")


# speedup vs baseline: 2.9210x; 2.9210x over previous
import functools

import jax
import jax.numpy as jnp
from jax import lax
from jax.experimental import pallas as pl
from jax.experimental.pallas import tpu as pltpu
from jax.experimental.pallas import tpu_sc as plsc

N = 10000
E = 320000
D = 128
H = 128
L = 3

NC = 2
NS = 16
NW = NC * NS
EPW = E // NW
K = 80
NCHUNK = EPW // K
NPAD = 10240
RPT = NPAD // NS
NJ = D // 16

_mesh = plsc.VectorSubcoreMesh(core_axis_name="c", subcore_axis_name="s")


@functools.partial(
    pl.kernel,
    out_type=jax.ShapeDtypeStruct((NC, NPAD, D), jnp.float32),
    mesh=_mesh,
    scratch_types=[
        pltpu.VMEM((K,), jnp.int32),
        pltpu.VMEM((K,), jnp.int32),
        pltpu.VMEM((K,), jnp.float32),
        pltpu.VMEM((K, D), jnp.float32),
        pltpu.VMEM_SHARED((NPAD, D), jnp.float32),
        pltpu.SemaphoreType.DMA,
    ],
)
def _sc_gather_scatter(h_hbm, src_hbm, dst_hbm, w_hbm, zero_hbm, out_hbm,
                       sidx, didx, wv, rows, acc, sem):
    cid = lax.axis_index("c")
    sid = lax.axis_index("s")
    wid = sid * NC + cid
    r0 = sid * RPT
    pltpu.sync_copy(zero_hbm.at[pl.ds(r0, RPT)], acc.at[pl.ds(r0, RPT)])
    plsc.subcore_barrier()
    base0 = wid * EPW

    def chunk(g, carry):
        base = base0 + g * K
        pltpu.sync_copy(src_hbm.at[pl.ds(base, K)], sidx)
        pltpu.sync_copy(w_hbm.at[pl.ds(base, K)], wv)
        pltpu.async_copy(h_hbm.at[sidx], rows, sem).wait()

        def scale_group(g2, c2):
            i0 = g2 * 16
            wvec = wv[pl.ds(i0, 16)]
            for r in range(16):
                i = i0 + r
                s = wvec[r]
                for j in range(NJ):
                    sl = pl.ds(j * 16, 16)
                    rows[i, sl] = rows[i, sl] * s
            return c2

        lax.fori_loop(0, K // 16, scale_group, 0)
        pltpu.sync_copy(dst_hbm.at[pl.ds(base, K)], didx)
        pltpu.sync_copy(rows, acc.at[didx], add=True)
        return carry

    lax.fori_loop(0, NCHUNK, chunk, 0)
    plsc.subcore_barrier()
    pltpu.sync_copy(acc.at[pl.ds(r0, RPT)], out_hbm.at[cid, pl.ds(r0, RPT)])


@functools.partial(
    pl.kernel,
    out_type=jax.ShapeDtypeStruct((E, D), jnp.float32),
    mesh=_mesh,
    scratch_types=[
        pltpu.VMEM((K,), jnp.int32),
        pltpu.VMEM((K,), jnp.int32),
        pltpu.VMEM((K,), jnp.float32),
        pltpu.VMEM((K, D), jnp.float32),
        pltpu.VMEM((K, D), jnp.float32),
        pltpu.VMEM((D,), jnp.float32),
        pltpu.SemaphoreType.DMA,
        pltpu.SemaphoreType.DMA,
    ],
)
def _sc_edge_feat(a_hbm, b_hbm, c_hbm, src_hbm, dst_hbm, w_hbm, out_hbm,
                  sidx, didx, wv, ra, rb, cv, sem_a, sem_b):
    cid = lax.axis_index("c")
    sid = lax.axis_index("s")
    wid = sid * NC + cid
    pltpu.sync_copy(c_hbm, cv)
    cvals = [cv[pl.ds(j * 16, 16)] for j in range(NJ)]
    base0 = wid * EPW

    def chunk(g, carry):
        base = base0 + g * K
        pltpu.sync_copy(src_hbm.at[pl.ds(base, K)], sidx)
        pltpu.sync_copy(dst_hbm.at[pl.ds(base, K)], didx)
        pltpu.sync_copy(w_hbm.at[pl.ds(base, K)], wv)
        cp_a = pltpu.async_copy(a_hbm.at[sidx], ra, sem_a)
        cp_b = pltpu.async_copy(b_hbm.at[didx], rb, sem_b)
        cp_a.wait()
        cp_b.wait()

        def row_group(g2, c2):
            i0 = g2 * 16
            wvec = wv[pl.ds(i0, 16)]
            for r in range(16):
                i = i0 + r
                s = wvec[r]
                for j in range(NJ):
                    sl = pl.ds(j * 16, 16)
                    ra[i, sl] = ra[i, sl] + rb[i, sl] + s * cvals[j]
            return c2

        lax.fori_loop(0, K // 16, row_group, 0)
        pltpu.sync_copy(ra, out_hbm.at[pl.ds(base, K)])
        return carry

    lax.fori_loop(0, NCHUNK, chunk, 0)


_INV_SQRT2 = 0.7071067811865476


def _gelu(x):
    return 0.5 * x * (1.0 + lax.erf(x * _INV_SQRT2))

def _dense_body(a0_ref, a1_ref, h_ref, wr_ref, br_ref, wro_ref, g_ref, b_ref,
                o_ref):
    agg = a0_ref[...] + a1_ref[...]
    y = jnp.dot(agg, wr_ref[...], preferred_element_type=jnp.float32)
    y = y + jnp.dot(h_ref[...], wro_ref[...], preferred_element_type=jnp.float32)
    y = y + br_ref[...]
    y = _gelu(y)
    m = jnp.mean(y, axis=0, keepdims=True)
    v = jnp.mean((y - m) ** 2, axis=0, keepdims=True)
    o_ref[...] = (y - m) / jnp.sqrt(v + 1e-5) * g_ref[...] + b_ref[...]


_tc_dense = pl.pallas_call(
    _dense_body,
    out_shape=jax.ShapeDtypeStruct((N, D), jnp.float32),
)


def _prep_body(h_ref, wa_ref, wb_ref, b1_ref, a_ref, b_ref):
    h = h_ref[...]
    a_ref[...] = jnp.dot(h, wa_ref[...], preferred_element_type=jnp.float32) + b1_ref[...]
    b_ref[...] = jnp.dot(h, wb_ref[...], preferred_element_type=jnp.float32)


_tc_prep = pl.pallas_call(
    _prep_body,
    out_shape=(jax.ShapeDtypeStruct((N, D), jnp.float32),
               jax.ShapeDtypeStruct((N, D), jnp.float32)),
)


def _post_body(pre_ref, w2_ref, b2_ref, o_ref):
    z = _gelu(pre_ref[...])
    t = jnp.dot(z, w2_ref[...], preferred_element_type=jnp.float32) + b2_ref[...]
    o_ref[...] = jax.nn.sigmoid(t)


_EBLK = 4000

_tc_post = pl.pallas_call(
    _post_body,
    grid=(E // _EBLK,),
    in_specs=[
        pl.BlockSpec((_EBLK, D), lambda i: (i, 0)),
        pl.BlockSpec((D, 1), lambda i: (0, 0)),
        pl.BlockSpec((1, 1), lambda i: (0, 0)),
    ],
    out_specs=pl.BlockSpec((_EBLK, 1), lambda i: (i, 0)),
    out_shape=jax.ShapeDtypeStruct((E, 1), jnp.float32),
)


def kernel(x, edge_index, edge_attr, W_rel, b_rel, W_root, gamma, beta,
           W1, b1, W2, b2):
    src = edge_index[0]
    dst = edge_index[1]
    w = edge_attr[:, 0]
    zeros = jnp.zeros((NPAD, D), jnp.float32)
    h = x
    for i in range(L):
        agg2 = _sc_gather_scatter(h, src, dst, w, zeros)
        h = _tc_dense(agg2[0, :N], agg2[1, :N], h, W_rel[i], b_rel[i].reshape(1, H),
                      W_root[i], gamma[i].reshape(1, H), beta[i].reshape(1, H))
    A, B = _tc_prep(h, W1[:H], W1[H:2 * H], b1.reshape(1, H))
    pre = _sc_edge_feat(A, B, W1[2 * H], src, dst, w)
    out = _tc_post(pre, W2, b2.reshape(1, 1))
    return out.reshape(E)


# speedup vs baseline: 4.9275x; 1.6869x over previous
import functools

import jax
import jax.numpy as jnp
from jax import lax
from jax.experimental import pallas as pl
from jax.experimental.pallas import tpu as pltpu
from jax.experimental.pallas import tpu_sc as plsc

N = 10000
E = 320000
D = 128
H = 128
L = 3

NC = 2
NS = 16
NW = NC * NS
EPW = E // NW
K = 40
NCHUNK = EPW // K
NBUF = 5
NSUPER = NCHUNK // NBUF
KL = 16
NCHUNKL = EPW // KL
NSUPERL = NCHUNKL // NBUF
NPAD = 10240
RPT = NPAD // NS
NJ = D // 16

_mesh = plsc.VectorSubcoreMesh(core_axis_name="c", subcore_axis_name="s")


@functools.partial(
    pl.kernel,
    out_type=jax.ShapeDtypeStruct((NC, NPAD, D), jnp.float32),
    mesh=_mesh,
    scratch_types=[
        pltpu.VMEM((2, NBUF, KL), jnp.int32),
        pltpu.VMEM((2, NBUF, KL), jnp.int32),
        pltpu.VMEM((2, NBUF, KL), jnp.float32),
        pltpu.VMEM((NBUF * KL, D), jnp.float32),
        pltpu.VMEM_SHARED((NPAD, D), jnp.float32),
    ] + [pltpu.SemaphoreType.DMA] * (2 * NBUF + 2),
)
def _sc_gather_scatter(h_hbm, src_hbm, dst_hbm, w_hbm, zero_hbm, out_hbm,
                       sidx, didx, wv, rows, acc, *sems):
    semg = sems[:NBUF]
    sems_s = sems[NBUF:2 * NBUF]
    sem_i = sems[2 * NBUF:]
    cid = lax.axis_index("c")
    sid = lax.axis_index("s")
    wid = sid * NC + cid
    r0 = sid * RPT
    pltpu.sync_copy(zero_hbm.at[pl.ds(r0, RPT)], acc.at[pl.ds(r0, RPT)])

    def _idx_cps(gg, slot):
        return [
            pltpu.make_async_copy(src_hbm.at[wid, gg], sidx.at[slot],
                                  sem_i[slot]),
            pltpu.make_async_copy(dst_hbm.at[wid, gg], didx.at[slot],
                                  sem_i[slot]),
            pltpu.make_async_copy(w_hbm.at[wid, gg], wv.at[slot],
                                  sem_i[slot]),
        ]

    def _rbuf(b):
        return rows.at[pl.ds(b * KL, KL)]

    def _scat_start(slot, b):
        pltpu.async_copy(_rbuf(b), acc.at[didx.at[slot, b]], sems_s[b],
                         add=True)

    def _scat_wait(slot, b):
        pltpu.make_async_copy(_rbuf(b), acc.at[didx.at[slot, b]],
                              sems_s[b]).wait()

    for cp in _idx_cps(0, 0):
        cp.start()
    plsc.subcore_barrier()

    def one_super(gg, slot, first, last):
        for cp in _idx_cps(gg, slot):
            cp.wait()
        gcps = []
        for b in range(NBUF):
            if first:
                @pl.when(gg > 0)
                def _wait(slot=slot, b=b):
                    _scat_wait(1 - slot, b)
            else:
                _scat_wait(1 - slot, b)
            gcps.append(
                pltpu.async_copy(h_hbm.at[sidx.at[slot, b]], _rbuf(b),
                                 semg[b]))
        if not last:
            @pl.when(gg + 1 < NSUPERL)
            def _prefetch(gg=gg, slot=slot):
                for cp in _idx_cps(gg + 1, 1 - slot):
                    cp.start()
        for b in range(NBUF):
            gcps[b].wait()
            wvec = wv[slot, b, :]
            for r in range(16):
                i = b * KL + r
                s = wvec[r]
                for j in range(NJ):
                    sl = pl.ds(j * 16, 16)
                    rows[i, sl] = rows[i, sl] * s
            _scat_start(slot, b)

    def super2(gg2, carry):
        one_super(2 * gg2, 0, True, False)
        one_super(2 * gg2 + 1, 1, False, False)
        return carry

    lax.fori_loop(0, NSUPERL // 2, super2, 0)
    one_super(NSUPERL - 1, 0, False, True)
    for b in range(NBUF):
        _scat_wait(0, b)
    plsc.subcore_barrier()
    pltpu.sync_copy(acc.at[pl.ds(r0, RPT)], out_hbm.at[cid, pl.ds(r0, RPT)])


@functools.partial(
    pl.kernel,
    out_type=jax.ShapeDtypeStruct((E, D), jnp.float32),
    mesh=_mesh,
    scratch_types=[
        pltpu.VMEM((EPW,), jnp.int32),
        pltpu.VMEM((EPW,), jnp.int32),
        pltpu.VMEM((NBUF * K, D), jnp.float32),
        pltpu.VMEM((NBUF * K, D), jnp.float32),
    ] + [pltpu.SemaphoreType.DMA] * (3 * NBUF),
)
def _sc_edge_feat(a_hbm, b_hbm, src_hbm, dst_hbm, out_hbm,
                  sidx, didx, ra, rb, *sems):
    sem_a = sems[:NBUF]
    sem_b = sems[NBUF:2 * NBUF]
    sem_o = sems[2 * NBUF:]
    cid = lax.axis_index("c")
    sid = lax.axis_index("s")
    wid = sid * NC + cid
    pltpu.sync_copy(src_hbm.at[wid], sidx)
    pltpu.sync_copy(dst_hbm.at[wid], didx)
    base0 = wid * EPW

    def _ostore(g, b):
        return pltpu.make_async_copy(ra.at[pl.ds(b * K, K)],
                                     out_hbm.at[pl.ds(base0 + g * K, K)],
                                     sem_o[b])

    def super_chunk(gg, carry):
        g0 = gg * NBUF
        acps, bcps = [], []
        for b in range(NBUF):
            @pl.when(gg > 0)
            def _wait(b=b):
                _ostore(g0 - NBUF + b, b).wait()
            acps.append(
                pltpu.async_copy(a_hbm.at[sidx.at[pl.ds((g0 + b) * K, K)]],
                                 ra.at[pl.ds(b * K, K)], sem_a[b]))
            bcps.append(
                pltpu.async_copy(b_hbm.at[didx.at[pl.ds((g0 + b) * K, K)]],
                                 rb.at[pl.ds(b * K, K)], sem_b[b]))
        for b in range(NBUF):
            g = g0 + b
            acps[b].wait()
            bcps[b].wait()

            def row_add(i0, c2, b=b):
                i = b * K + i0
                for j in range(NJ):
                    sl = pl.ds(j * 16, 16)
                    ra[i, sl] = ra[i, sl] + rb[i, sl]
                return c2

            lax.fori_loop(0, K, row_add, 0)
            _ostore(g, b).start()
        return carry

    lax.fori_loop(0, NSUPER, super_chunk, 0)
    for b in range(NBUF):
        _ostore(NCHUNK - NBUF + b, b).wait()


_INV_SQRT2 = 0.7071067811865476


def _gelu(x):
    return 0.5 * x * (1.0 + lax.erf(x * _INV_SQRT2))


def _dense_body(a0_ref, a1_ref, h_ref, wr_ref, br_ref, wro_ref, g_ref, b_ref,
                o_ref):
    agg = a0_ref[...] + a1_ref[...]
    y = jnp.dot(agg, wr_ref[...], preferred_element_type=jnp.float32)
    y = y + jnp.dot(h_ref[...], wro_ref[...], preferred_element_type=jnp.float32)
    y = y + br_ref[...]
    y = _gelu(y)
    m = jnp.mean(y, axis=0, keepdims=True)
    v = jnp.mean((y - m) ** 2, axis=0, keepdims=True)
    o_ref[...] = (y - m) / jnp.sqrt(v + 1e-5) * g_ref[...] + b_ref[...]


_tc_dense = pl.pallas_call(
    _dense_body,
    out_shape=jax.ShapeDtypeStruct((N, D), jnp.float32),
)


def _prep_body(h_ref, wa_ref, wb_ref, b1_ref, a_ref, b_ref):
    h = h_ref[...]
    a_ref[...] = jnp.dot(h, wa_ref[...], preferred_element_type=jnp.float32) + b1_ref[...]
    b_ref[...] = jnp.dot(h, wb_ref[...], preferred_element_type=jnp.float32)


_tc_prep = pl.pallas_call(
    _prep_body,
    out_shape=(jax.ShapeDtypeStruct((N, D), jnp.float32),
               jax.ShapeDtypeStruct((N, D), jnp.float32)),
)


def _post_body(pre_ref, w_ref, c_ref, w2_ref, b2_ref, o_ref):
    z = _gelu(pre_ref[...] + w_ref[...] * c_ref[...])
    t = jnp.dot(z, w2_ref[...], preferred_element_type=jnp.float32) + b2_ref[...]
    o_ref[...] = jax.nn.sigmoid(t)


_EBLK = 4000

_tc_post = pl.pallas_call(
    _post_body,
    grid=(E // _EBLK,),
    in_specs=[
        pl.BlockSpec((_EBLK, D), lambda i: (i, 0)),
        pl.BlockSpec((_EBLK, 1), lambda i: (i, 0)),
        pl.BlockSpec((1, D), lambda i: (0, 0)),
        pl.BlockSpec((D, 1), lambda i: (0, 0)),
        pl.BlockSpec((1, 1), lambda i: (0, 0)),
    ],
    out_specs=pl.BlockSpec((_EBLK, 1), lambda i: (i, 0)),
    out_shape=jax.ShapeDtypeStruct((E, 1), jnp.float32),
)


def kernel(x, edge_index, edge_attr, W_rel, b_rel, W_root, gamma, beta,
           W1, b1, W2, b2):
    src = edge_index[0]
    dst = edge_index[1]
    w = edge_attr[:, 0]
    src2 = src.reshape(NW, EPW)
    dst2 = dst.reshape(NW, EPW)
    src4 = src.reshape(NW, NSUPERL, NBUF, KL)
    dst4 = dst.reshape(NW, NSUPERL, NBUF, KL)
    w4 = w.reshape(NW, NSUPERL, NBUF, KL)
    zeros = jnp.zeros((NPAD, D), jnp.float32)
    h = x
    for i in range(L):
        agg2 = _sc_gather_scatter(h, src4, dst4, w4, zeros)
        h = _tc_dense(agg2[0, :N], agg2[1, :N], h, W_rel[i],
                      b_rel[i].reshape(1, H), W_root[i],
                      gamma[i].reshape(1, H), beta[i].reshape(1, H))
    A, B = _tc_prep(h, W1[:H], W1[H:2 * H], b1.reshape(1, H))
    pre = _sc_edge_feat(A, B, src2, dst2)
    out = _tc_post(pre, edge_attr, W1[2 * H].reshape(1, H), W2,
                   b2.reshape(1, 1))
    return out.reshape(E)


# speedup vs baseline: 5.9175x; 1.2009x over previous
import functools

import jax
import jax.numpy as jnp
from jax import lax
from jax.experimental import pallas as pl
from jax.experimental.pallas import tpu as pltpu
from jax.experimental.pallas import tpu_sc as plsc

N = 10000
E = 320000
D = 128
H = 128
L = 3

NC = 2
NS = 16
NW = NC * NS
EPW = E // NW
K = 40
NCHUNK = EPW // K
NBUF = 5
NSUPER = NCHUNK // NBUF
KL = 40
NCHUNKL = EPW // KL
NSUPERL = NCHUNKL // NBUF
NPAD = 10240
RPT = NPAD // NS
NJ = D // 16

_mesh = plsc.VectorSubcoreMesh(core_axis_name="c", subcore_axis_name="s")


@functools.partial(
    pl.kernel,
    out_type=jax.ShapeDtypeStruct((NC, NPAD, D), jnp.float32),
    mesh=_mesh,
    scratch_types=[
        pltpu.VMEM((2, NBUF, KL), jnp.int32),
        pltpu.VMEM((2, NBUF, KL), jnp.int32),
        pltpu.VMEM((2, NBUF, KL), jnp.float32),
        pltpu.VMEM((NBUF * KL, D), jnp.float32),
        pltpu.VMEM_SHARED((NPAD, D), jnp.float32),
    ] + [pltpu.SemaphoreType.DMA] * (2 * NBUF + 2),
)
def _sc_gather_scatter(h_hbm, src_hbm, dst_hbm, w_hbm, zero_hbm, out_hbm,
                       sidx, didx, wv, rows, acc, *sems):
    semg = sems[:NBUF]
    sems_s = sems[NBUF:2 * NBUF]
    sem_i = sems[2 * NBUF:]
    cid = lax.axis_index("c")
    sid = lax.axis_index("s")
    wid = sid * NC + cid
    r0 = sid * RPT
    pltpu.sync_copy(zero_hbm.at[pl.ds(r0, RPT)], acc.at[pl.ds(r0, RPT)])

    def _idx_cps(gg, slot):
        return [
            pltpu.make_async_copy(src_hbm.at[wid, gg], sidx.at[slot],
                                  sem_i[slot]),
            pltpu.make_async_copy(dst_hbm.at[wid, gg], didx.at[slot],
                                  sem_i[slot]),
            pltpu.make_async_copy(w_hbm.at[wid, gg], wv.at[slot],
                                  sem_i[slot]),
        ]

    def _rbuf(b):
        return rows.at[pl.ds(b * KL, KL)]

    def _scat_start(slot, b):
        pltpu.async_copy(_rbuf(b), acc.at[didx.at[slot, b]], sems_s[b],
                         add=True)

    def _scat_wait(slot, b):
        pltpu.make_async_copy(_rbuf(b), acc.at[didx.at[slot, b]],
                              sems_s[b]).wait()

    for cp in _idx_cps(0, 0):
        cp.start()
    plsc.subcore_barrier()

    def one_super(gg, slot, first, last):
        for cp in _idx_cps(gg, slot):
            cp.wait()
        gcps = []
        for b in range(NBUF):
            if first:
                @pl.when(gg > 0)
                def _wait(slot=slot, b=b):
                    _scat_wait(1 - slot, b)
            else:
                _scat_wait(1 - slot, b)
            gcps.append(
                pltpu.async_copy(h_hbm.at[sidx.at[slot, b]], _rbuf(b),
                                 semg[b]))
        if not last:
            @pl.when(gg + 1 < NSUPERL)
            def _prefetch(gg=gg, slot=slot):
                for cp in _idx_cps(gg + 1, 1 - slot):
                    cp.start()
        for b in range(NBUF):
            gcps[b].wait()

            def scale_group(g2, c2, b=b, slot=slot):
                i0 = g2 * 16
                wvec = wv[slot, b, pl.ds(i0, 16)]
                for r in range(16):
                    i = b * KL + i0 + r
                    s = wvec[r]
                    for j in range(NJ):
                        sl = pl.ds(j * 16, 16)
                        rows[i, sl] = rows[i, sl] * s
                return c2

            lax.fori_loop(0, KL // 16, scale_group, 0)
            wtail = wv[slot, b, pl.ds(KL - 16, 16)]
            for r in range(8):
                i = b * KL + (KL - 8) + r
                s = wtail[r + 8]
                for j in range(NJ):
                    sl = pl.ds(j * 16, 16)
                    rows[i, sl] = rows[i, sl] * s
            _scat_start(slot, b)

    def super2(gg2, carry):
        one_super(2 * gg2, 0, True, False)
        one_super(2 * gg2 + 1, 1, False, False)
        return carry

    lax.fori_loop(0, NSUPERL // 2, super2, 0)
    for b in range(NBUF):
        _scat_wait(1, b)
    plsc.subcore_barrier()
    pltpu.sync_copy(acc.at[pl.ds(r0, RPT)], out_hbm.at[cid, pl.ds(r0, RPT)])


@functools.partial(
    pl.kernel,
    out_type=jax.ShapeDtypeStruct((E, D), jnp.float32),
    mesh=_mesh,
    scratch_types=[
        pltpu.VMEM((EPW,), jnp.int32),
        pltpu.VMEM((EPW,), jnp.int32),
        pltpu.VMEM((NBUF * K, D), jnp.float32),
        pltpu.VMEM((NBUF * K, D), jnp.float32),
    ] + [pltpu.SemaphoreType.DMA] * (3 * NBUF),
)
def _sc_edge_feat(a_hbm, b_hbm, src_hbm, dst_hbm, out_hbm,
                  sidx, didx, ra, rb, *sems):
    sem_a = sems[:NBUF]
    sem_b = sems[NBUF:2 * NBUF]
    sem_o = sems[2 * NBUF:]
    cid = lax.axis_index("c")
    sid = lax.axis_index("s")
    wid = sid * NC + cid
    pltpu.sync_copy(src_hbm.at[wid], sidx)
    pltpu.sync_copy(dst_hbm.at[wid], didx)
    base0 = wid * EPW

    def _ostore(g, b):
        return pltpu.make_async_copy(ra.at[pl.ds(b * K, K)],
                                     out_hbm.at[pl.ds(base0 + g * K, K)],
                                     sem_o[b])

    def super_chunk(gg, carry):
        g0 = gg * NBUF
        acps, bcps = [], []
        for b in range(NBUF):
            @pl.when(gg > 0)
            def _wait(b=b):
                _ostore(g0 - NBUF + b, b).wait()
            acps.append(
                pltpu.async_copy(a_hbm.at[sidx.at[pl.ds((g0 + b) * K, K)]],
                                 ra.at[pl.ds(b * K, K)], sem_a[b]))
            bcps.append(
                pltpu.async_copy(b_hbm.at[didx.at[pl.ds((g0 + b) * K, K)]],
                                 rb.at[pl.ds(b * K, K)], sem_b[b]))
        for b in range(NBUF):
            g = g0 + b
            acps[b].wait()
            bcps[b].wait()

            def row_add(i0, c2, b=b):
                i = b * K + i0
                for j in range(NJ):
                    sl = pl.ds(j * 16, 16)
                    ra[i, sl] = ra[i, sl] + rb[i, sl]
                return c2

            lax.fori_loop(0, K, row_add, 0)
            _ostore(g, b).start()
        return carry

    lax.fori_loop(0, NSUPER, super_chunk, 0)
    for b in range(NBUF):
        _ostore(NCHUNK - NBUF + b, b).wait()


_INV_SQRT2 = 0.7071067811865476


def _gelu(x):
    return 0.5 * x * (1.0 + lax.erf(x * _INV_SQRT2))


def _dense_body(a0_ref, a1_ref, h_ref, wr_ref, br_ref, wro_ref, g_ref, b_ref,
                o_ref):
    agg = a0_ref[...] + a1_ref[...]
    y = jnp.dot(agg, wr_ref[...], preferred_element_type=jnp.float32)
    y = y + jnp.dot(h_ref[...], wro_ref[...], preferred_element_type=jnp.float32)
    y = y + br_ref[...]
    y = _gelu(y)
    m = jnp.mean(y, axis=0, keepdims=True)
    v = jnp.mean((y - m) ** 2, axis=0, keepdims=True)
    o_ref[...] = (y - m) / jnp.sqrt(v + 1e-5) * g_ref[...] + b_ref[...]


_tc_dense = pl.pallas_call(
    _dense_body,
    out_shape=jax.ShapeDtypeStruct((N, D), jnp.float32),
)


def _dense_final_body(a0_ref, a1_ref, h_ref, wr_ref, br_ref, wro_ref, g_ref,
                      b_ref, wa_ref, wb_ref, b1_ref, a_ref, bb_ref):
    agg = a0_ref[...] + a1_ref[...]
    y = jnp.dot(agg, wr_ref[...], preferred_element_type=jnp.float32)
    y = y + jnp.dot(h_ref[...], wro_ref[...], preferred_element_type=jnp.float32)
    y = y + br_ref[...]
    y = _gelu(y)
    m = jnp.mean(y, axis=0, keepdims=True)
    v = jnp.mean((y - m) ** 2, axis=0, keepdims=True)
    h3 = (y - m) / jnp.sqrt(v + 1e-5) * g_ref[...] + b_ref[...]
    a_ref[...] = jnp.dot(h3, wa_ref[...], preferred_element_type=jnp.float32) + b1_ref[...]
    bb_ref[...] = jnp.dot(h3, wb_ref[...], preferred_element_type=jnp.float32)


_tc_dense_final = pl.pallas_call(
    _dense_final_body,
    out_shape=(jax.ShapeDtypeStruct((N, D), jnp.float32),
               jax.ShapeDtypeStruct((N, D), jnp.float32)),
)


def _post_body(pre_ref, w_ref, c_ref, w2_ref, b2_ref, o_ref):
    z = _gelu(pre_ref[...] + w_ref[...] * c_ref[...])
    t = jnp.dot(z, w2_ref[...], preferred_element_type=jnp.float32) + b2_ref[...]
    o_ref[...] = jax.nn.sigmoid(t)


_EBLK = 4000

_tc_post = pl.pallas_call(
    _post_body,
    grid=(E // _EBLK,),
    in_specs=[
        pl.BlockSpec((_EBLK, D), lambda i: (i, 0)),
        pl.BlockSpec((_EBLK, 1), lambda i: (i, 0)),
        pl.BlockSpec((1, D), lambda i: (0, 0)),
        pl.BlockSpec((D, 1), lambda i: (0, 0)),
        pl.BlockSpec((1, 1), lambda i: (0, 0)),
    ],
    out_specs=pl.BlockSpec((_EBLK, 1), lambda i: (i, 0)),
    out_shape=jax.ShapeDtypeStruct((E, 1), jnp.float32),
)


def kernel(x, edge_index, edge_attr, W_rel, b_rel, W_root, gamma, beta,
           W1, b1, W2, b2):
    src = edge_index[0]
    dst = edge_index[1]
    w = edge_attr[:, 0]
    src2 = src.reshape(NW, EPW)
    dst2 = dst.reshape(NW, EPW)
    src4 = src.reshape(NW, NSUPERL, NBUF, KL)
    dst4 = dst.reshape(NW, NSUPERL, NBUF, KL)
    w4 = w.reshape(NW, NSUPERL, NBUF, KL)
    zeros = jnp.zeros((NPAD, D), jnp.float32)
    h = x
    for i in range(L - 1):
        agg2 = _sc_gather_scatter(h, src4, dst4, w4, zeros)
        h = _tc_dense(agg2[0, :N], agg2[1, :N], h, W_rel[i],
                      b_rel[i].reshape(1, H), W_root[i],
                      gamma[i].reshape(1, H), beta[i].reshape(1, H))
    agg2 = _sc_gather_scatter(h, src4, dst4, w4, zeros)
    A, B = _tc_dense_final(agg2[0, :N], agg2[1, :N], h, W_rel[L - 1],
                           b_rel[L - 1].reshape(1, H), W_root[L - 1],
                           gamma[L - 1].reshape(1, H),
                           beta[L - 1].reshape(1, H),
                           W1[:H], W1[H:2 * H], b1.reshape(1, H))
    pre = _sc_edge_feat(A, B, src2, dst2)
    out = _tc_post(pre, edge_attr, W1[2 * H].reshape(1, H), W2,
                   b2.reshape(1, 1))
    return out.reshape(E)


# speedup vs baseline: 5.9302x; 1.0022x over previous
import functools

import jax
import jax.numpy as jnp
from jax import lax
from jax.experimental import pallas as pl
from jax.experimental.pallas import tpu as pltpu
from jax.experimental.pallas import tpu_sc as plsc

N = 10000
E = 320000
D = 128
H = 128
L = 3

NC = 2
NS = 16
NW = NC * NS
EPW = E // NW
K = 40
NCHUNK = EPW // K
NBUF = 5
NSUPER = NCHUNK // NBUF
KL = 40
NCHUNKL = EPW // KL
NSUPERL = NCHUNKL // NBUF
NPAD = 10240
RPT = NPAD // NS
NJ = D // 16

_mesh = plsc.VectorSubcoreMesh(core_axis_name="c", subcore_axis_name="s")


@functools.partial(
    pl.kernel,
    out_type=jax.ShapeDtypeStruct((NC, NPAD, D), jnp.float32),
    mesh=_mesh,
    scratch_types=[
        pltpu.VMEM((2, NBUF, KL), jnp.int32),
        pltpu.VMEM((2, NBUF, KL), jnp.int32),
        pltpu.VMEM((2, NBUF, KL), jnp.float32),
        pltpu.VMEM((NBUF * KL, D), jnp.float32),
        pltpu.VMEM_SHARED((NPAD, D), jnp.float32),
    ] + [pltpu.SemaphoreType.DMA] * (2 * NBUF + 2),
)
def _sc_gather_scatter(h_hbm, src_hbm, dst_hbm, w_hbm, zero_hbm, out_hbm,
                       sidx, didx, wv, rows, acc, *sems):
    semg = sems[:NBUF]
    sems_s = sems[NBUF:2 * NBUF]
    sem_i = sems[2 * NBUF:]
    cid = lax.axis_index("c")
    sid = lax.axis_index("s")
    wid = sid * NC + cid
    r0 = sid * RPT
    pltpu.sync_copy(zero_hbm.at[pl.ds(r0, RPT)], acc.at[pl.ds(r0, RPT)])

    def _idx_cps(gg, slot):
        return [
            pltpu.make_async_copy(src_hbm.at[wid, gg], sidx.at[slot],
                                  sem_i[slot]),
            pltpu.make_async_copy(dst_hbm.at[wid, gg], didx.at[slot],
                                  sem_i[slot]),
            pltpu.make_async_copy(w_hbm.at[wid, gg], wv.at[slot],
                                  sem_i[slot]),
        ]

    def _rbuf(b):
        return rows.at[pl.ds(b * KL, KL)]

    def _scat_start(slot, b):
        pltpu.async_copy(_rbuf(b), acc.at[didx.at[slot, b]], sems_s[b],
                         add=True)

    def _scat_wait(slot, b):
        pltpu.make_async_copy(_rbuf(b), acc.at[didx.at[slot, b]],
                              sems_s[b]).wait()

    for cp in _idx_cps(0, 0):
        cp.start()
    plsc.subcore_barrier()

    def one_super(gg, slot, first, last):
        for cp in _idx_cps(gg, slot):
            cp.wait()
        gcps = []
        for b in range(NBUF):
            if first:
                @pl.when(gg > 0)
                def _wait(slot=slot, b=b):
                    _scat_wait(1 - slot, b)
            else:
                _scat_wait(1 - slot, b)
            gcps.append(
                pltpu.async_copy(h_hbm.at[sidx.at[slot, b]], _rbuf(b),
                                 semg[b]))
        if not last:
            @pl.when(gg + 1 < NSUPERL)
            def _prefetch(gg=gg, slot=slot):
                for cp in _idx_cps(gg + 1, 1 - slot):
                    cp.start()
        for b in range(NBUF):
            gcps[b].wait()

            def scale_group(g2, c2, b=b, slot=slot):
                i0 = g2 * 16
                wvec = wv[slot, b, pl.ds(i0, 16)]
                for r in range(16):
                    i = b * KL + i0 + r
                    s = wvec[r]
                    for j in range(NJ):
                        sl = pl.ds(j * 16, 16)
                        rows[i, sl] = rows[i, sl] * s
                return c2

            lax.fori_loop(0, KL // 16, scale_group, 0)
            wtail = wv[slot, b, pl.ds(KL - 16, 16)]
            for r in range(8):
                i = b * KL + (KL - 8) + r
                s = wtail[r + 8]
                for j in range(NJ):
                    sl = pl.ds(j * 16, 16)
                    rows[i, sl] = rows[i, sl] * s
            _scat_start(slot, b)

    def super2(gg2, carry):
        one_super(2 * gg2, 0, True, False)
        one_super(2 * gg2 + 1, 1, False, False)
        return carry

    lax.fori_loop(0, NSUPERL // 2, super2, 0)
    for b in range(NBUF):
        _scat_wait(1, b)
    plsc.subcore_barrier()
    pltpu.sync_copy(acc.at[pl.ds(r0, RPT)], out_hbm.at[cid, pl.ds(r0, RPT)])


@functools.partial(
    pl.kernel,
    out_type=jax.ShapeDtypeStruct((E, D), jnp.float32),
    mesh=_mesh,
    scratch_types=[
        pltpu.VMEM((EPW,), jnp.int32),
        pltpu.VMEM((EPW,), jnp.int32),
        pltpu.VMEM((NBUF * K, D), jnp.float32),
        pltpu.VMEM((NBUF * K, D), jnp.float32),
    ] + [pltpu.SemaphoreType.DMA] * (3 * NBUF),
)
def _sc_edge_feat(a_hbm, b_hbm, src_hbm, dst_hbm, out_hbm,
                  sidx, didx, ra, rb, *sems):
    sem_a = sems[:NBUF]
    sem_b = sems[NBUF:2 * NBUF]
    sem_o = sems[2 * NBUF:]
    cid = lax.axis_index("c")
    sid = lax.axis_index("s")
    wid = sid * NC + cid
    pltpu.sync_copy(src_hbm.at[wid], sidx)
    pltpu.sync_copy(dst_hbm.at[wid], didx)
    base0 = wid * EPW

    def _ostore(g, b):
        return pltpu.make_async_copy(ra.at[pl.ds(b * K, K)],
                                     out_hbm.at[pl.ds(base0 + g * K, K)],
                                     sem_o[b])

    def super_chunk(gg, carry):
        g0 = gg * NBUF
        acps, bcps = [], []
        for b in range(NBUF):
            @pl.when(gg > 0)
            def _wait(b=b):
                _ostore(g0 - NBUF + b, b).wait()
            acps.append(
                pltpu.async_copy(a_hbm.at[sidx.at[pl.ds((g0 + b) * K, K)]],
                                 ra.at[pl.ds(b * K, K)], sem_a[b]))
            bcps.append(
                pltpu.async_copy(b_hbm.at[didx.at[pl.ds((g0 + b) * K, K)]],
                                 rb.at[pl.ds(b * K, K)], sem_b[b]))
        for b in range(NBUF):
            g = g0 + b
            acps[b].wait()
            bcps[b].wait()

            def row_add(i0, c2, b=b):
                i = b * K + i0
                for j in range(NJ):
                    sl = pl.ds(j * 16, 16)
                    ra[i, sl] = ra[i, sl] + rb[i, sl]
                return c2

            lax.fori_loop(0, K, row_add, 0)
            _ostore(g, b).start()
        return carry

    lax.fori_loop(0, NSUPER, super_chunk, 0)
    for b in range(NBUF):
        _ostore(NCHUNK - NBUF + b, b).wait()


_INV_SQRT2 = 0.7071067811865476


def _gelu(x):
    return 0.5 * x * (1.0 + lax.erf(x * _INV_SQRT2))


def _dense_body(a0_ref, a1_ref, h_ref, wr_ref, br_ref, wro_ref, g_ref, b_ref,
                o_ref):
    agg = a0_ref[...] + a1_ref[...]
    y = jnp.dot(agg, wr_ref[...], preferred_element_type=jnp.float32)
    y = y + jnp.dot(h_ref[...], wro_ref[...], preferred_element_type=jnp.float32)
    y = y + br_ref[...]
    y = _gelu(y)
    m = jnp.mean(y, axis=0, keepdims=True)
    v = jnp.mean((y - m) ** 2, axis=0, keepdims=True)
    o_ref[...] = (y - m) / jnp.sqrt(v + 1e-5) * g_ref[...] + b_ref[...]


_tc_dense = pl.pallas_call(
    _dense_body,
    out_shape=jax.ShapeDtypeStruct((N, D), jnp.float32),
)


def _dense_final_body(a0_ref, a1_ref, h_ref, wr_ref, br_ref, wro_ref, g_ref,
                      b_ref, wa_ref, wb_ref, b1_ref, a_ref, bb_ref):
    agg = a0_ref[...] + a1_ref[...]
    y = jnp.dot(agg, wr_ref[...], preferred_element_type=jnp.float32)
    y = y + jnp.dot(h_ref[...], wro_ref[...], preferred_element_type=jnp.float32)
    y = y + br_ref[...]
    y = _gelu(y)
    m = jnp.mean(y, axis=0, keepdims=True)
    v = jnp.mean((y - m) ** 2, axis=0, keepdims=True)
    h3 = (y - m) / jnp.sqrt(v + 1e-5) * g_ref[...] + b_ref[...]
    a_ref[...] = jnp.dot(h3, wa_ref[...], preferred_element_type=jnp.float32) + b1_ref[...]
    bb_ref[...] = jnp.dot(h3, wb_ref[...], preferred_element_type=jnp.float32)


_tc_dense_final = pl.pallas_call(
    _dense_final_body,
    out_shape=(jax.ShapeDtypeStruct((N, D), jnp.float32),
               jax.ShapeDtypeStruct((N, D), jnp.float32)),
)


def _post_body(pre_ref, w_ref, c_ref, w2_ref, b2_ref, o_ref):
    x = pre_ref[...] + w_ref[...] * c_ref[...]
    z = 0.5 * x * (1.0 + jnp.tanh(0.7978845608028654 * (x + 0.044715 * x * x * x)))
    t = jnp.dot(z, w2_ref[...], preferred_element_type=jnp.float32) + b2_ref[...]
    o_ref[...] = jax.nn.sigmoid(t)


_EBLK = 8000

_tc_post = pl.pallas_call(
    _post_body,
    grid=(E // _EBLK,),
    in_specs=[
        pl.BlockSpec((_EBLK, D), lambda i: (i, 0)),
        pl.BlockSpec((_EBLK, 1), lambda i: (i, 0)),
        pl.BlockSpec((1, D), lambda i: (0, 0)),
        pl.BlockSpec((D, 1), lambda i: (0, 0)),
        pl.BlockSpec((1, 1), lambda i: (0, 0)),
    ],
    out_specs=pl.BlockSpec((_EBLK, 1), lambda i: (i, 0)),
    out_shape=jax.ShapeDtypeStruct((E, 1), jnp.float32),
)


def kernel(x, edge_index, edge_attr, W_rel, b_rel, W_root, gamma, beta,
           W1, b1, W2, b2):
    src = edge_index[0]
    dst = edge_index[1]
    w = edge_attr[:, 0]
    src2 = src.reshape(NW, EPW)
    dst2 = dst.reshape(NW, EPW)
    src4 = src.reshape(NW, NSUPERL, NBUF, KL)
    dst4 = dst.reshape(NW, NSUPERL, NBUF, KL)
    w4 = w.reshape(NW, NSUPERL, NBUF, KL)
    zeros = jnp.zeros((NPAD, D), jnp.float32)
    h = x
    for i in range(L - 1):
        agg2 = _sc_gather_scatter(h, src4, dst4, w4, zeros)
        h = _tc_dense(agg2[0, :N], agg2[1, :N], h, W_rel[i],
                      b_rel[i].reshape(1, H), W_root[i],
                      gamma[i].reshape(1, H), beta[i].reshape(1, H))
    agg2 = _sc_gather_scatter(h, src4, dst4, w4, zeros)
    A, B = _tc_dense_final(agg2[0, :N], agg2[1, :N], h, W_rel[L - 1],
                           b_rel[L - 1].reshape(1, H), W_root[L - 1],
                           gamma[L - 1].reshape(1, H),
                           beta[L - 1].reshape(1, H),
                           W1[:H], W1[H:2 * H], b1.reshape(1, H))
    pre = _sc_edge_feat(A, B, src2, dst2)
    out = _tc_post(pre, edge_attr, W1[2 * H].reshape(1, H), W2,
                   b2.reshape(1, 1))
    return out.reshape(E)


# speedup vs baseline: 6.8129x; 1.1488x over previous
import functools

import jax
import jax.numpy as jnp
from jax import lax
from jax.experimental import pallas as pl
from jax.experimental.pallas import tpu as pltpu
from jax.experimental.pallas import tpu_sc as plsc

N = 10000
E = 320000
D = 128
H = 128
L = 3

NC = 2
NS = 16
NW = NC * NS
EPW = E // NW
K = 40
NCHUNK = EPW // K
NBUF = 5
NSUPER = NCHUNK // NBUF
KL = 40
NCHUNKL = EPW // KL
NSUPERL = NCHUNKL // NBUF
NPAD = 10240
RPT = NPAD // NS
NJ = D // 16

_mesh = plsc.VectorSubcoreMesh(core_axis_name="c", subcore_axis_name="s")


@functools.partial(
    pl.kernel,
    out_type=jax.ShapeDtypeStruct((NC, NPAD, D), jnp.float32),
    mesh=_mesh,
    scratch_types=[
        pltpu.VMEM((2, NBUF, KL), jnp.int32),
        pltpu.VMEM((2, NBUF, KL), jnp.int32),
        pltpu.VMEM((2, NBUF, KL), jnp.float32),
        pltpu.VMEM((NBUF * KL, D), jnp.float32),
        pltpu.VMEM_SHARED((NPAD, D), jnp.float32),
    ] + [pltpu.SemaphoreType.DMA] * (2 * NBUF + 2),
)
def _sc_gather_scatter(h_hbm, src_hbm, dst_hbm, w_hbm, zero_hbm, out_hbm,
                       sidx, didx, wv, rows, acc, *sems):
    semg = sems[:NBUF]
    sems_s = sems[NBUF:2 * NBUF]
    sem_i = sems[2 * NBUF:]
    cid = lax.axis_index("c")
    sid = lax.axis_index("s")
    wid = sid * NC + cid
    r0 = sid * RPT
    pltpu.sync_copy(zero_hbm.at[pl.ds(r0, RPT)], acc.at[pl.ds(r0, RPT)])

    def _idx_cps(gg, slot):
        return [
            pltpu.make_async_copy(src_hbm.at[wid, gg], sidx.at[slot],
                                  sem_i[slot]),
            pltpu.make_async_copy(dst_hbm.at[wid, gg], didx.at[slot],
                                  sem_i[slot]),
            pltpu.make_async_copy(w_hbm.at[wid, gg], wv.at[slot],
                                  sem_i[slot]),
        ]

    def _rbuf(b):
        return rows.at[pl.ds(b * KL, KL)]

    def _scat_start(slot, b):
        pltpu.async_copy(_rbuf(b), acc.at[didx.at[slot, b]], sems_s[b],
                         add=True)

    def _scat_wait(slot, b):
        pltpu.make_async_copy(_rbuf(b), acc.at[didx.at[slot, b]],
                              sems_s[b]).wait()

    for cp in _idx_cps(0, 0):
        cp.start()
    plsc.subcore_barrier()

    def one_super(gg, slot, first, last):
        for cp in _idx_cps(gg, slot):
            cp.wait()
        gcps = []
        for b in range(NBUF):
            if first:
                @pl.when(gg > 0)
                def _wait(slot=slot, b=b):
                    _scat_wait(1 - slot, b)
            else:
                _scat_wait(1 - slot, b)
            gcps.append(
                pltpu.async_copy(h_hbm.at[sidx.at[slot, b]], _rbuf(b),
                                 semg[b]))
        if not last:
            @pl.when(gg + 1 < NSUPERL)
            def _prefetch(gg=gg, slot=slot):
                for cp in _idx_cps(gg + 1, 1 - slot):
                    cp.start()
        for b in range(NBUF):
            gcps[b].wait()

            def scale_group(g2, c2, b=b, slot=slot):
                i0 = g2 * 16
                wvec = wv[slot, b, pl.ds(i0, 16)]
                for r in range(16):
                    i = b * KL + i0 + r
                    s = wvec[r]
                    for j in range(NJ):
                        sl = pl.ds(j * 16, 16)
                        rows[i, sl] = rows[i, sl] * s
                return c2

            lax.fori_loop(0, KL // 16, scale_group, 0)
            wtail = wv[slot, b, pl.ds(KL - 16, 16)]
            for r in range(8):
                i = b * KL + (KL - 8) + r
                s = wtail[r + 8]
                for j in range(NJ):
                    sl = pl.ds(j * 16, 16)
                    rows[i, sl] = rows[i, sl] * s
            _scat_start(slot, b)

    def super2(gg2, carry):
        one_super(2 * gg2, 0, True, False)
        one_super(2 * gg2 + 1, 1, False, False)
        return carry

    lax.fori_loop(0, NSUPERL // 2, super2, 0)
    for b in range(NBUF):
        _scat_wait(1, b)
    plsc.subcore_barrier()
    pltpu.sync_copy(acc.at[pl.ds(r0, RPT)], out_hbm.at[cid, pl.ds(r0, RPT)])


@functools.partial(
    pl.kernel,
    out_type=jax.ShapeDtypeStruct((E, D), jnp.float32),
    mesh=_mesh,
    scratch_types=[
        pltpu.VMEM((EPW,), jnp.int32),
        pltpu.VMEM((EPW,), jnp.int32),
        pltpu.VMEM((NBUF * K, D), jnp.float32),
        pltpu.VMEM((NBUF * K, D), jnp.float32),
    ] + [pltpu.SemaphoreType.DMA] * (3 * NBUF),
)
def _sc_edge_feat(a_hbm, b_hbm, src_hbm, dst_hbm, out_hbm,
                  sidx, didx, ra, rb, *sems):
    sem_a = sems[:NBUF]
    sem_b = sems[NBUF:2 * NBUF]
    sem_o = sems[2 * NBUF:]
    cid = lax.axis_index("c")
    sid = lax.axis_index("s")
    wid = sid * NC + cid
    pltpu.sync_copy(src_hbm.at[wid], sidx)
    pltpu.sync_copy(dst_hbm.at[wid], didx)
    base0 = wid * EPW

    def _ostore(g, b):
        return pltpu.make_async_copy(ra.at[pl.ds(b * K, K)],
                                     out_hbm.at[pl.ds(base0 + g * K, K)],
                                     sem_o[b])

    def super_chunk(gg, carry):
        g0 = gg * NBUF
        acps, bcps = [], []
        for b in range(NBUF):
            @pl.when(gg > 0)
            def _wait(b=b):
                _ostore(g0 - NBUF + b, b).wait()
            acps.append(
                pltpu.async_copy(a_hbm.at[sidx.at[pl.ds((g0 + b) * K, K)]],
                                 ra.at[pl.ds(b * K, K)], sem_a[b]))
            bcps.append(
                pltpu.async_copy(b_hbm.at[didx.at[pl.ds((g0 + b) * K, K)]],
                                 rb.at[pl.ds(b * K, K)], sem_b[b]))
        for b in range(NBUF):
            g = g0 + b
            acps[b].wait()
            bcps[b].wait()

            def row_add(i0, c2, b=b):
                i = b * K + i0
                for j in range(NJ):
                    sl = pl.ds(j * 16, 16)
                    ra[i, sl] = ra[i, sl] + rb[i, sl]
                return c2

            lax.fori_loop(0, K, row_add, 0)
            _ostore(g, b).start()
        return carry

    lax.fori_loop(0, NSUPER, super_chunk, 0)
    for b in range(NBUF):
        _ostore(NCHUNK - NBUF + b, b).wait()


_INV_SQRT2 = 0.7071067811865476


def _gelu(x):
    return 0.5 * x * (1.0 + lax.erf(x * _INV_SQRT2))


def _dense_body(agg_ref, h_ref, wr_ref, br_ref, wro_ref, g_ref, b_ref,
                o_ref):
    agg = agg_ref[0, :N] + agg_ref[1, :N]
    y = jnp.dot(agg, wr_ref[...], preferred_element_type=jnp.float32)
    y = y + jnp.dot(h_ref[...], wro_ref[...], preferred_element_type=jnp.float32)
    y = y + br_ref[...]
    y = _gelu(y)
    m = jnp.mean(y, axis=0, keepdims=True)
    v = jnp.mean((y - m) ** 2, axis=0, keepdims=True)
    o_ref[...] = (y - m) / jnp.sqrt(v + 1e-5) * g_ref[...] + b_ref[...]


_tc_dense = pl.pallas_call(
    _dense_body,
    out_shape=jax.ShapeDtypeStruct((N, D), jnp.float32),
)


def _dense_final_body(agg_ref, h_ref, wr_ref, br_ref, wro_ref, g_ref,
                      b_ref, wa_ref, wb_ref, b1_ref, a_ref, bb_ref):
    agg = agg_ref[0, :N] + agg_ref[1, :N]
    y = jnp.dot(agg, wr_ref[...], preferred_element_type=jnp.float32)
    y = y + jnp.dot(h_ref[...], wro_ref[...], preferred_element_type=jnp.float32)
    y = y + br_ref[...]
    y = _gelu(y)
    m = jnp.mean(y, axis=0, keepdims=True)
    v = jnp.mean((y - m) ** 2, axis=0, keepdims=True)
    h3 = (y - m) / jnp.sqrt(v + 1e-5) * g_ref[...] + b_ref[...]
    a_ref[...] = jnp.dot(h3, wa_ref[...], preferred_element_type=jnp.float32) + b1_ref[...]
    bb_ref[...] = jnp.dot(h3, wb_ref[...], preferred_element_type=jnp.float32)


_tc_dense_final = pl.pallas_call(
    _dense_final_body,
    out_shape=(jax.ShapeDtypeStruct((N, D), jnp.float32),
               jax.ShapeDtypeStruct((N, D), jnp.float32)),
)


def _post_body(pre_ref, w_ref, c_ref, w2_ref, b2_ref, o_ref):
    i = pl.program_id(0)
    x = pre_ref[...] + w_ref[pl.ds(i * _EBLK, _EBLK)][:, None] * c_ref[...]
    z = 0.5 * x * (1.0 + jnp.tanh(0.7978845608028654 * (x + 0.044715 * x * x * x)))
    t = jnp.dot(z, w2_ref[...], preferred_element_type=jnp.float32)
    o_ref[...] = jax.nn.sigmoid(t.reshape(_EBLK // D, D) + b2_ref[0, 0])


_EBLK = 16384
_EPAD = 20 * _EBLK

_tc_post = pl.pallas_call(
    _post_body,
    grid=(_EPAD // _EBLK,),
    in_specs=[
        pl.BlockSpec((_EBLK, D), lambda i: (i, 0)),
        pl.BlockSpec((_EPAD,), lambda i: (0,)),
        pl.BlockSpec((1, D), lambda i: (0, 0)),
        pl.BlockSpec((D, 1), lambda i: (0, 0)),
        pl.BlockSpec((1, 1), lambda i: (0, 0)),
    ],
    out_specs=pl.BlockSpec((_EBLK // D, D), lambda i: (i, 0)),
    out_shape=jax.ShapeDtypeStruct((_EPAD // D, D), jnp.float32),
)


def kernel(x, edge_index, edge_attr, W_rel, b_rel, W_root, gamma, beta,
           W1, b1, W2, b2):
    src = edge_index[0]
    dst = edge_index[1]
    w = edge_attr[:, 0]
    src2 = src.reshape(NW, EPW)
    dst2 = dst.reshape(NW, EPW)
    src4 = src.reshape(NW, NSUPERL, NBUF, KL)
    dst4 = dst.reshape(NW, NSUPERL, NBUF, KL)
    w4 = w.reshape(NW, NSUPERL, NBUF, KL)
    zeros = jnp.zeros((NPAD, D), jnp.float32)
    h = x
    for i in range(L - 1):
        agg2 = _sc_gather_scatter(h, src4, dst4, w4, zeros)
        h = _tc_dense(agg2, h, W_rel[i],
                      b_rel[i].reshape(1, H), W_root[i],
                      gamma[i].reshape(1, H), beta[i].reshape(1, H))
    agg2 = _sc_gather_scatter(h, src4, dst4, w4, zeros)
    A, B = _tc_dense_final(agg2, h, W_rel[L - 1],
                           b_rel[L - 1].reshape(1, H), W_root[L - 1],
                           gamma[L - 1].reshape(1, H),
                           beta[L - 1].reshape(1, H),
                           W1[:H], W1[H:2 * H], b1.reshape(1, H))
    pre = _sc_edge_feat(A, B, src2, dst2)
    w_pad = jnp.pad(w, (0, _EPAD - E))
    out = _tc_post(pre, w_pad, W1[2 * H].reshape(1, H), W2, b2.reshape(1, 1))
    return out.reshape(_EPAD)[:E]


# speedup vs baseline: 6.8270x; 1.0021x over previous
import functools

import jax
import jax.numpy as jnp
from jax import lax
from jax.experimental import pallas as pl
from jax.experimental.pallas import tpu as pltpu
from jax.experimental.pallas import tpu_sc as plsc

N = 10000
E = 320000
D = 128
H = 128
L = 3

NC = 2
NS = 16
NW = NC * NS
EPW = E // NW
K = 40
NCHUNK = EPW // K
NBUF = 5
NSUPER = NCHUNK // NBUF
KL = 40
NCHUNKL = EPW // KL
NSUPERL = NCHUNKL // NBUF
NPAD = 10240
RPT = NPAD // NS
NJ = D // 16

_mesh = plsc.VectorSubcoreMesh(core_axis_name="c", subcore_axis_name="s")


@functools.partial(
    pl.kernel,
    out_type=jax.ShapeDtypeStruct((NC, NPAD, D), jnp.float32),
    mesh=_mesh,
    scratch_types=[
        pltpu.VMEM((2, NBUF, KL), jnp.int32),
        pltpu.VMEM((2, NBUF, KL), jnp.int32),
        pltpu.VMEM((2, NBUF, KL), jnp.float32),
        pltpu.VMEM((NBUF * KL, D), jnp.float32),
        pltpu.VMEM_SHARED((NPAD, D), jnp.float32),
    ] + [pltpu.SemaphoreType.DMA] * (2 * NBUF + 2),
)
def _sc_gather_scatter(h_hbm, src_hbm, dst_hbm, w_hbm, zero_hbm, out_hbm,
                       sidx, didx, wv, rows, acc, *sems):
    semg = sems[:NBUF]
    sems_s = sems[NBUF:2 * NBUF]
    sem_i = sems[2 * NBUF:]
    cid = lax.axis_index("c")
    sid = lax.axis_index("s")
    wid = sid * NC + cid
    r0 = sid * RPT
    pltpu.sync_copy(zero_hbm.at[pl.ds(r0, RPT)], acc.at[pl.ds(r0, RPT)])

    def _idx_cps(gg, slot):
        return [
            pltpu.make_async_copy(src_hbm.at[wid, gg], sidx.at[slot],
                                  sem_i[slot]),
            pltpu.make_async_copy(dst_hbm.at[wid, gg], didx.at[slot],
                                  sem_i[slot]),
            pltpu.make_async_copy(w_hbm.at[wid, gg], wv.at[slot],
                                  sem_i[slot]),
        ]

    def _rbuf(b):
        return rows.at[pl.ds(b * KL, KL)]

    def _scat_start(slot, b):
        pltpu.async_copy(_rbuf(b), acc.at[didx.at[slot, b]], sems_s[b],
                         add=True)

    def _scat_wait(slot, b):
        pltpu.make_async_copy(_rbuf(b), acc.at[didx.at[slot, b]],
                              sems_s[b]).wait()

    for cp in _idx_cps(0, 0):
        cp.start()
    plsc.subcore_barrier()

    def one_super(gg, slot, first, last):
        for cp in _idx_cps(gg, slot):
            cp.wait()
        gcps = []
        for b in range(NBUF):
            if first:
                @pl.when(gg > 0)
                def _wait(slot=slot, b=b):
                    _scat_wait(1 - slot, b)
            else:
                _scat_wait(1 - slot, b)
            gcps.append(
                pltpu.async_copy(h_hbm.at[sidx.at[slot, b]], _rbuf(b),
                                 semg[b]))
        if not last:
            @pl.when(gg + 1 < NSUPERL)
            def _prefetch(gg=gg, slot=slot):
                for cp in _idx_cps(gg + 1, 1 - slot):
                    cp.start()
        for b in range(NBUF):
            gcps[b].wait()

            def scale_group(g2, c2, b=b, slot=slot):
                i0 = g2 * 16
                wvec = wv[slot, b, pl.ds(i0, 16)]
                for r in range(16):
                    i = b * KL + i0 + r
                    s = wvec[r]
                    for j in range(NJ):
                        sl = pl.ds(j * 16, 16)
                        rows[i, sl] = rows[i, sl] * s
                return c2

            lax.fori_loop(0, KL // 16, scale_group, 0)
            wtail = wv[slot, b, pl.ds(KL - 16, 16)]
            for r in range(8):
                i = b * KL + (KL - 8) + r
                s = wtail[r + 8]
                for j in range(NJ):
                    sl = pl.ds(j * 16, 16)
                    rows[i, sl] = rows[i, sl] * s
            _scat_start(slot, b)

    def super2(gg2, carry):
        one_super(2 * gg2, 0, True, False)
        one_super(2 * gg2 + 1, 1, False, False)
        return carry

    lax.fori_loop(0, NSUPERL // 2, super2, 0)
    for b in range(NBUF):
        _scat_wait(1, b)
    plsc.subcore_barrier()
    pltpu.sync_copy(acc.at[pl.ds(r0, RPT)], out_hbm.at[cid, pl.ds(r0, RPT)])


@functools.partial(
    pl.kernel,
    out_type=jax.ShapeDtypeStruct((E, D), jnp.float32),
    mesh=_mesh,
    scratch_types=[
        pltpu.VMEM((EPW,), jnp.int32),
        pltpu.VMEM((EPW,), jnp.int32),
        pltpu.VMEM((NBUF * K, D), jnp.float32),
        pltpu.VMEM((NBUF * K, D), jnp.float32),
    ] + [pltpu.SemaphoreType.DMA] * (3 * NBUF),
)
def _sc_edge_feat(a_hbm, b_hbm, src_hbm, dst_hbm, out_hbm,
                  sidx, didx, ra, rb, *sems):
    sem_a = sems[:NBUF]
    sem_b = sems[NBUF:2 * NBUF]
    sem_o = sems[2 * NBUF:]
    cid = lax.axis_index("c")
    sid = lax.axis_index("s")
    wid = sid * NC + cid
    pltpu.sync_copy(src_hbm.at[wid], sidx)
    pltpu.sync_copy(dst_hbm.at[wid], didx)
    base0 = wid * EPW

    def _ostore(g, b):
        return pltpu.make_async_copy(ra.at[pl.ds(b * K, K)],
                                     out_hbm.at[pl.ds(base0 + g * K, K)],
                                     sem_o[b])

    def super_chunk(gg, carry):
        g0 = gg * NBUF
        acps, bcps = [], []
        for b in range(NBUF):
            @pl.when(gg > 0)
            def _wait(b=b):
                _ostore(g0 - NBUF + b, b).wait()
            acps.append(
                pltpu.async_copy(a_hbm.at[sidx.at[pl.ds((g0 + b) * K, K)]],
                                 ra.at[pl.ds(b * K, K)], sem_a[b]))
            bcps.append(
                pltpu.async_copy(b_hbm.at[didx.at[pl.ds((g0 + b) * K, K)]],
                                 rb.at[pl.ds(b * K, K)], sem_b[b]))
        for b in range(NBUF):
            g = g0 + b
            acps[b].wait()
            bcps[b].wait()

            def row_add(i0, c2, b=b):
                i = b * K + i0
                for j in range(NJ):
                    sl = pl.ds(j * 16, 16)
                    ra[i, sl] = ra[i, sl] + rb[i, sl]
                return c2

            lax.fori_loop(0, K, row_add, 0)
            _ostore(g, b).start()
        return carry

    lax.fori_loop(0, NSUPER, super_chunk, 0)
    for b in range(NBUF):
        _ostore(NCHUNK - NBUF + b, b).wait()


_INV_SQRT2 = 0.7071067811865476


def _gelu(x):
    return 0.5 * x * (1.0 + lax.erf(x * _INV_SQRT2))


def _dense_body(agg_ref, h_ref, wr_ref, br_ref, wro_ref, g_ref, b_ref,
                o_ref):
    agg = agg_ref[0, :N] + agg_ref[1, :N]
    y = jnp.dot(agg, wr_ref[...], preferred_element_type=jnp.float32)
    y = y + jnp.dot(h_ref[...], wro_ref[...], preferred_element_type=jnp.float32)
    y = y + br_ref[...]
    y = _gelu(y)
    m = jnp.mean(y, axis=0, keepdims=True)
    v = jnp.mean((y - m) ** 2, axis=0, keepdims=True)
    o_ref[...] = (y - m) / jnp.sqrt(v + 1e-5) * g_ref[...] + b_ref[...]


_tc_dense = pl.pallas_call(
    _dense_body,
    out_shape=jax.ShapeDtypeStruct((N, D), jnp.float32),
)


def _dense_final_body(agg_ref, h_ref, wr_ref, br_ref, wro_ref, g_ref,
                      b_ref, wa_ref, wb_ref, b1_ref, a_ref, bb_ref):
    agg = agg_ref[0, :N] + agg_ref[1, :N]
    y = jnp.dot(agg, wr_ref[...], preferred_element_type=jnp.float32)
    y = y + jnp.dot(h_ref[...], wro_ref[...], preferred_element_type=jnp.float32)
    y = y + br_ref[...]
    y = _gelu(y)
    m = jnp.mean(y, axis=0, keepdims=True)
    v = jnp.mean((y - m) ** 2, axis=0, keepdims=True)
    h3 = (y - m) / jnp.sqrt(v + 1e-5) * g_ref[...] + b_ref[...]
    a = jnp.dot(h3, wa_ref[...], preferred_element_type=jnp.float32) + b1_ref[...]
    bb = jnp.dot(h3, wb_ref[...], preferred_element_type=jnp.float32)
    a_ref[...] = a
    bb_ref[...] = bb


_tc_dense_final = pl.pallas_call(
    _dense_final_body,
    out_shape=(jax.ShapeDtypeStruct((N, D), jnp.float32),
               jax.ShapeDtypeStruct((N, D), jnp.float32)),
)


def _post_body(pre_ref, w_ref, c_ref, w2_ref, b2_ref, o_ref):
    i = pl.program_id(0)
    x = pre_ref[...] + w_ref[pl.ds(i * _EBLK, _EBLK)][:, None] * c_ref[...]
    z = 0.5 * x * (1.0 + jnp.tanh(0.7978845608028654 * (x + 0.044715 * x * x * x)))
    t = jnp.dot(z, w2_ref[...], preferred_element_type=jnp.float32)
    o_ref[...] = jax.nn.sigmoid(t.reshape(_EBLK // D, D) + b2_ref[0, 0])


_EBLK = 16384
_EPAD = 20 * _EBLK

_tc_post = pl.pallas_call(
    _post_body,
    grid=(_EPAD // _EBLK,),
    in_specs=[
        pl.BlockSpec((_EBLK, D), lambda i: (i, 0)),
        pl.BlockSpec((_EPAD,), lambda i: (0,)),
        pl.BlockSpec((1, D), lambda i: (0, 0)),
        pl.BlockSpec((D, 1), lambda i: (0, 0)),
        pl.BlockSpec((1, 1), lambda i: (0, 0)),
    ],
    out_specs=pl.BlockSpec((_EBLK // D, D), lambda i: (i, 0)),
    out_shape=jax.ShapeDtypeStruct((_EPAD // D, D), jnp.float32),
)


def kernel(x, edge_index, edge_attr, W_rel, b_rel, W_root, gamma, beta,
           W1, b1, W2, b2):
    src = edge_index[0]
    dst = edge_index[1]
    w = edge_attr[:, 0]
    src2 = src.reshape(NW, EPW)
    dst2 = dst.reshape(NW, EPW)
    src4 = src.reshape(NW, NSUPERL, NBUF, KL)
    dst4 = dst.reshape(NW, NSUPERL, NBUF, KL)
    w4 = w.reshape(NW, NSUPERL, NBUF, KL)
    zeros = jnp.zeros((NPAD, D), jnp.float32)
    h = x
    for i in range(L - 1):
        agg2 = _sc_gather_scatter(h, src4, dst4, w4, zeros)
        h = _tc_dense(agg2, h, W_rel[i],
                      b_rel[i].reshape(1, H), W_root[i],
                      gamma[i].reshape(1, H), beta[i].reshape(1, H))
    agg2 = _sc_gather_scatter(h, src4, dst4, w4, zeros)
    A, B = _tc_dense_final(agg2, h, W_rel[L - 1],
                           b_rel[L - 1].reshape(1, H), W_root[L - 1],
                           gamma[L - 1].reshape(1, H),
                           beta[L - 1].reshape(1, H),
                           W1[:H], W1[H:2 * H], b1.reshape(1, H))
    pre = _sc_edge_feat(A, B, src2, dst2)
    w_pad = jnp.pad(w, (0, _EPAD - E))
    out = _tc_post(pre, w_pad, W1[2 * H].reshape(1, H), W2, b2.reshape(1, 1))
    return out.reshape(_EPAD)[:E]


# speedup vs baseline: 7.0441x; 1.0318x over previous
import functools

import jax
import jax.numpy as jnp
from jax import lax
from jax.experimental import pallas as pl
from jax.experimental.pallas import tpu as pltpu
from jax.experimental.pallas import tpu_sc as plsc

N = 10000
E = 320000
D = 128
H = 128
L = 3

NC = 2
NS = 16
NW = NC * NS
EPW = E // NW
K = 40
NCHUNK = EPW // K
NBUF = 5
NSUPER = NCHUNK // NBUF
KL = 40
NCHUNKL = EPW // KL
NSUPERL = NCHUNKL // NBUF
NPAD = 10240
RPT = NPAD // NS
NJ = D // 16

_mesh = plsc.VectorSubcoreMesh(core_axis_name="c", subcore_axis_name="s")


@functools.partial(
    pl.kernel,
    out_type=jax.ShapeDtypeStruct((NC, NPAD, D), jnp.float32),
    mesh=_mesh,
    scratch_types=[
        pltpu.VMEM((2, NBUF, KL), jnp.int32),
        pltpu.VMEM((2, NBUF, KL), jnp.int32),
        pltpu.VMEM((2, NBUF, KL), jnp.float32),
        pltpu.VMEM((NBUF * KL, D), jnp.float32),
        pltpu.VMEM_SHARED((NPAD, D), jnp.float32),
    ] + [pltpu.SemaphoreType.DMA] * (2 * NBUF + 2),
)
def _sc_gather_scatter(h_hbm, src_hbm, dst_hbm, w_hbm, zero_hbm, out_hbm,
                       sidx, didx, wv, rows, acc, *sems):
    semg = sems[:NBUF]
    sems_s = sems[NBUF:2 * NBUF]
    sem_i = sems[2 * NBUF:]
    cid = lax.axis_index("c")
    sid = lax.axis_index("s")
    wid = sid * NC + cid
    r0 = sid * RPT
    pltpu.sync_copy(zero_hbm.at[pl.ds(r0, RPT)], acc.at[pl.ds(r0, RPT)])

    def _idx_cps(gg, slot):
        return [
            pltpu.make_async_copy(src_hbm.at[wid, gg], sidx.at[slot],
                                  sem_i[slot]),
            pltpu.make_async_copy(dst_hbm.at[wid, gg], didx.at[slot],
                                  sem_i[slot]),
            pltpu.make_async_copy(w_hbm.at[wid, gg], wv.at[slot],
                                  sem_i[slot]),
        ]

    def _rbuf(b):
        return rows.at[pl.ds(b * KL, KL)]

    def _scat_start(slot, b):
        pltpu.async_copy(_rbuf(b), acc.at[didx.at[slot, b]], sems_s[b],
                         add=True)

    def _scat_wait(slot, b):
        pltpu.make_async_copy(_rbuf(b), acc.at[didx.at[slot, b]],
                              sems_s[b]).wait()

    for cp in _idx_cps(0, 0):
        cp.start()
    plsc.subcore_barrier()

    def one_super(gg, slot, first, last):
        for cp in _idx_cps(gg, slot):
            cp.wait()
        gcps = []
        for b in range(NBUF):
            if first:
                @pl.when(gg > 0)
                def _wait(slot=slot, b=b):
                    _scat_wait(1 - slot, b)
            else:
                _scat_wait(1 - slot, b)
            gcps.append(
                pltpu.async_copy(h_hbm.at[sidx.at[slot, b]], _rbuf(b),
                                 semg[b]))
        if not last:
            @pl.when(gg + 1 < NSUPERL)
            def _prefetch(gg=gg, slot=slot):
                for cp in _idx_cps(gg + 1, 1 - slot):
                    cp.start()
        for b in range(NBUF):
            gcps[b].wait()

            def scale_group(g2, c2, b=b, slot=slot):
                i0 = g2 * 16
                wvec = wv[slot, b, pl.ds(i0, 16)]
                for r in range(16):
                    i = b * KL + i0 + r
                    s = wvec[r]
                    for j in range(NJ):
                        sl = pl.ds(j * 16, 16)
                        rows[i, sl] = rows[i, sl] * s
                return c2

            lax.fori_loop(0, KL // 16, scale_group, 0)
            wtail = wv[slot, b, pl.ds(KL - 16, 16)]
            for r in range(8):
                i = b * KL + (KL - 8) + r
                s = wtail[r + 8]
                for j in range(NJ):
                    sl = pl.ds(j * 16, 16)
                    rows[i, sl] = rows[i, sl] * s
            _scat_start(slot, b)

    def super2(gg2, carry):
        one_super(2 * gg2, 0, True, False)
        one_super(2 * gg2 + 1, 1, False, False)
        return carry

    lax.fori_loop(0, NSUPERL // 2, super2, 0)
    for b in range(NBUF):
        _scat_wait(1, b)
    plsc.subcore_barrier()
    pltpu.sync_copy(acc.at[pl.ds(r0, RPT)], out_hbm.at[cid, pl.ds(r0, RPT)])


EPW_E = EPW // 2
NCHUNK_E = EPW_E // K
NSUPER_E = NCHUNK_E // NBUF
EHALF = E // 2


@functools.partial(
    pl.kernel,
    out_type=jax.ShapeDtypeStruct((EHALF, D), jnp.float32),
    mesh=_mesh,
    scratch_types=[
        pltpu.VMEM((EPW_E,), jnp.int32),
        pltpu.VMEM((EPW_E,), jnp.int32),
        pltpu.VMEM((NBUF * K, D), jnp.float32),
        pltpu.VMEM((NBUF * K, D), jnp.float32),
    ] + [pltpu.SemaphoreType.DMA] * (3 * NBUF),
)
def _sc_edge_feat(a_hbm, b_hbm, src_hbm, dst_hbm, out_hbm,
                  sidx, didx, ra, rb, *sems):
    sem_a = sems[:NBUF]
    sem_b = sems[NBUF:2 * NBUF]
    sem_o = sems[2 * NBUF:]
    cid = lax.axis_index("c")
    sid = lax.axis_index("s")
    wid = sid * NC + cid
    pltpu.sync_copy(src_hbm.at[wid], sidx)
    pltpu.sync_copy(dst_hbm.at[wid], didx)
    base0 = wid * EPW_E

    def _ostore(g, b):
        return pltpu.make_async_copy(ra.at[pl.ds(b * K, K)],
                                     out_hbm.at[pl.ds(base0 + g * K, K)],
                                     sem_o[b])

    def super_chunk(gg, carry):
        g0 = gg * NBUF
        acps, bcps = [], []
        for b in range(NBUF):
            @pl.when(gg > 0)
            def _wait(b=b):
                _ostore(g0 - NBUF + b, b).wait()
            acps.append(
                pltpu.async_copy(a_hbm.at[sidx.at[pl.ds((g0 + b) * K, K)]],
                                 ra.at[pl.ds(b * K, K)], sem_a[b]))
            bcps.append(
                pltpu.async_copy(b_hbm.at[didx.at[pl.ds((g0 + b) * K, K)]],
                                 rb.at[pl.ds(b * K, K)], sem_b[b]))
        for b in range(NBUF):
            g = g0 + b
            acps[b].wait()
            bcps[b].wait()

            def row_add(i0, c2, b=b):
                i = b * K + i0
                for j in range(NJ):
                    sl = pl.ds(j * 16, 16)
                    ra[i, sl] = ra[i, sl] + rb[i, sl]
                return c2

            lax.fori_loop(0, K, row_add, 0)
            _ostore(g, b).start()
        return carry

    lax.fori_loop(0, NSUPER_E, super_chunk, 0)
    for b in range(NBUF):
        _ostore(NCHUNK_E - NBUF + b, b).wait()


_INV_SQRT2 = 0.7071067811865476


def _gelu(x):
    return 0.5 * x * (1.0 + lax.erf(x * _INV_SQRT2))


def _dense_body(agg_ref, h_ref, wr_ref, br_ref, wro_ref, g_ref, b_ref,
                o_ref):
    agg = agg_ref[0, :N] + agg_ref[1, :N]
    y = jnp.dot(agg, wr_ref[...], preferred_element_type=jnp.float32)
    y = y + jnp.dot(h_ref[...], wro_ref[...], preferred_element_type=jnp.float32)
    y = y + br_ref[...]
    y = _gelu(y)
    m = jnp.mean(y, axis=0, keepdims=True)
    v = jnp.mean((y - m) ** 2, axis=0, keepdims=True)
    o_ref[...] = (y - m) / jnp.sqrt(v + 1e-5) * g_ref[...] + b_ref[...]


_tc_dense = pl.pallas_call(
    _dense_body,
    out_shape=jax.ShapeDtypeStruct((N, D), jnp.float32),
)


def _dense_final_body(agg_ref, h_ref, wr_ref, br_ref, wro_ref, g_ref,
                      b_ref, wa_ref, wb_ref, b1_ref, a_ref, bb_ref):
    agg = agg_ref[0, :N] + agg_ref[1, :N]
    y = jnp.dot(agg, wr_ref[...], preferred_element_type=jnp.float32)
    y = y + jnp.dot(h_ref[...], wro_ref[...], preferred_element_type=jnp.float32)
    y = y + br_ref[...]
    y = _gelu(y)
    m = jnp.mean(y, axis=0, keepdims=True)
    v = jnp.mean((y - m) ** 2, axis=0, keepdims=True)
    h3 = (y - m) / jnp.sqrt(v + 1e-5) * g_ref[...] + b_ref[...]
    a = jnp.dot(h3, wa_ref[...], preferred_element_type=jnp.float32) + b1_ref[...]
    bb = jnp.dot(h3, wb_ref[...], preferred_element_type=jnp.float32)
    a_ref[...] = a
    bb_ref[...] = bb


_tc_dense_final = pl.pallas_call(
    _dense_final_body,
    out_shape=(jax.ShapeDtypeStruct((N, D), jnp.float32),
               jax.ShapeDtypeStruct((N, D), jnp.float32)),
)


def _post_body(pre_ref, w_ref, c_ref, w2_ref, b2_ref, o_ref):
    i = pl.program_id(0)
    x = pre_ref[...] + w_ref[pl.ds(i * _EBLK, _EBLK)][:, None] * c_ref[...]
    z = 0.5 * x * (1.0 + jnp.tanh(0.7978845608028654 * (x + 0.044715 * x * x * x)))
    t = jnp.dot(z, w2_ref[...], preferred_element_type=jnp.float32)
    o_ref[...] = jax.nn.sigmoid(t.reshape(_EBLK // D, D) + b2_ref[0, 0])


_EBLK = 16384
_EPAD = 10 * _EBLK

_tc_post = pl.pallas_call(
    _post_body,
    grid=(_EPAD // _EBLK,),
    in_specs=[
        pl.BlockSpec((_EBLK, D), lambda i: (i, 0)),
        pl.BlockSpec((_EPAD,), lambda i: (0,)),
        pl.BlockSpec((1, D), lambda i: (0, 0)),
        pl.BlockSpec((D, 1), lambda i: (0, 0)),
        pl.BlockSpec((1, 1), lambda i: (0, 0)),
    ],
    out_specs=pl.BlockSpec((_EBLK // D, D), lambda i: (i, 0)),
    out_shape=jax.ShapeDtypeStruct((_EPAD // D, D), jnp.float32),
)


def kernel(x, edge_index, edge_attr, W_rel, b_rel, W_root, gamma, beta,
           W1, b1, W2, b2):
    src = edge_index[0]
    dst = edge_index[1]
    w = edge_attr[:, 0]
    src4 = src.reshape(NW, NSUPERL, NBUF, KL)
    dst4 = dst.reshape(NW, NSUPERL, NBUF, KL)
    w4 = w.reshape(NW, NSUPERL, NBUF, KL)
    zeros = jnp.zeros((NPAD, D), jnp.float32)
    h = x
    for i in range(L - 1):
        agg2 = _sc_gather_scatter(h, src4, dst4, w4, zeros)
        h = _tc_dense(agg2, h, W_rel[i],
                      b_rel[i].reshape(1, H), W_root[i],
                      gamma[i].reshape(1, H), beta[i].reshape(1, H))
    agg2 = _sc_gather_scatter(h, src4, dst4, w4, zeros)
    A, B = _tc_dense_final(agg2, h, W_rel[L - 1],
                           b_rel[L - 1].reshape(1, H), W_root[L - 1],
                           gamma[L - 1].reshape(1, H),
                           beta[L - 1].reshape(1, H),
                           W1[:H], W1[H:2 * H], b1.reshape(1, H))
    srcH = src.reshape(NW, 2, EPW_E)
    dstH = dst.reshape(NW, 2, EPW_E)
    wH = w.reshape(NW, 2, EPW_E)
    c_row = W1[2 * H].reshape(1, H)
    b2_11 = b2.reshape(1, 1)
    halves = []
    for hh in (0, 1):
        pre = _sc_edge_feat(A, B, srcH[:, hh], dstH[:, hh])
        w_pad = jnp.pad(wH[:, hh].reshape(EHALF), (0, _EPAD - EHALF))
        o = _tc_post(pre, w_pad, c_row, W2, b2_11)
        halves.append(o.reshape(_EPAD)[:EHALF].reshape(NW, EPW_E))
    return jnp.stack(halves, axis=1).reshape(E)
